# Initial kernel scaffold; baseline (speedup 1.0000x reference)
import functools

import jax
import jax.numpy as jnp
from jax import lax
from jax.experimental import pallas as pl
from jax.experimental.pallas import tpu as pltpu
from jax.experimental.pallas import tpu_sc as plsc

N = 10000
E = 320000
D = 128

NC = 2
NS = 16
NW = NC * NS
C = 80
CHUNKS = E // C
CPW = CHUNKS // NW
RPS = N // NS
ZR = 125

_mesh = plsc.VectorSubcoreMesh(core_axis_name="c", subcore_axis_name="s")


@jax.jit
def _sc_degree(dst):
    @functools.partial(
        pl.kernel,
        out_type=jax.ShapeDtypeStruct((NC, N, 16), jnp.float32),
        mesh=_mesh,
        scratch_types=[
            pltpu.VMEM((C,), jnp.int32),
            pltpu.VMEM((C, 16), jnp.float32),
            pltpu.VMEM((RPS, 16), jnp.float32),
            pltpu.VMEM_SHARED((N, 16), jnp.float32),
        ],
    )
    def k(dst_hbm, out_hbm, idx_v, ones_v, zbuf_v, acc_sh):
        c = lax.axis_index("c")
        s = lax.axis_index("s")
        w = s * NC + c

        @pl.loop(0, C)
        def _(r):
            ones_v[r, :] = jnp.ones((16,), jnp.float32)

        @pl.loop(0, RPS)
        def _(r):
            zbuf_v[r, :] = jnp.zeros((16,), jnp.float32)

        pltpu.sync_copy(zbuf_v, acc_sh.at[pl.ds(s * RPS, RPS)])
        plsc.subcore_barrier()

        @pl.loop(0, CPW)
        def _(t):
            chunk = t * NW + w
            pltpu.sync_copy(dst_hbm.at[pl.ds(chunk * C, C)], idx_v)
            pltpu.sync_copy(ones_v, acc_sh.at[idx_v], add=True)

        plsc.subcore_barrier()
        pltpu.sync_copy(acc_sh.at[pl.ds(s * RPS, RPS)],
                        out_hbm.at[c, pl.ds(s * RPS, RPS)])

    return k(dst)


@jax.jit
def _sc_conv(g, src, dst):
    @functools.partial(
        pl.kernel,
        out_type=jax.ShapeDtypeStruct((NC, N, D), jnp.float32),
        mesh=_mesh,
        scratch_types=[
            pltpu.VMEM((C,), jnp.int32),
            pltpu.VMEM((C,), jnp.int32),
            pltpu.VMEM((C, D), jnp.float32),
            pltpu.VMEM((ZR, D), jnp.float32),
            pltpu.VMEM_SHARED((N, D), jnp.float32),
        ],
    )
    def k(g_hbm, src_hbm, dst_hbm, out_hbm, si_v, di_v, rows_v, zbuf_v, acc_sh):
        c = lax.axis_index("c")
        s = lax.axis_index("s")
        w = s * NC + c

        @pl.loop(0, ZR)
        def _(r):
            @pl.loop(0, D // 16)
            def _(gcol):
                zbuf_v[r, pl.ds(gcol * 16, 16)] = jnp.zeros((16,), jnp.float32)

        @pl.loop(0, RPS // ZR)
        def _(j):
            pltpu.sync_copy(zbuf_v, acc_sh.at[pl.ds(s * RPS + j * ZR, ZR)])

        plsc.subcore_barrier()

        @pl.loop(0, CPW)
        def _(t):
            chunk = t * NW + w
            base = chunk * C
            pltpu.sync_copy(src_hbm.at[pl.ds(base, C)], si_v)
            pltpu.sync_copy(dst_hbm.at[pl.ds(base, C)], di_v)
            pltpu.sync_copy(g_hbm.at[si_v], rows_v)
            pltpu.sync_copy(rows_v, acc_sh.at[di_v], add=True)

        plsc.subcore_barrier()
        pltpu.sync_copy(acc_sh.at[pl.ds(s * RPS, RPS)],
                        out_hbm.at[c, pl.ds(s * RPS, RPS)])

    return k(g, src, dst)


def _mm_t(a, w):
    return lax.dot_general(a, w, (((1,), (1,)), ((), ())),
                           preferred_element_type=jnp.float32)


@jax.jit
def _tc_pre(x, W_pre, b_pre2, W1):
    def body(x_ref, wp_ref, bp_ref, w1_ref, o_ref):
        h = _mm_t(x_ref[...], wp_ref[...]) + bp_ref[...]
        o_ref[...] = _mm_t(h, w1_ref[...])
    return pl.pallas_call(
        body, out_shape=jax.ShapeDtypeStruct((N, D), jnp.float32),
    )(x, W_pre, b_pre2, W1)


@jax.jit
def _tc_norms(degp, m1):
    def body(p_ref, m_ref, g_ref, dv_ref):
        deg = 1.0 + p_ref[0, :, :1] + p_ref[1, :, :1]
        dinv = lax.rsqrt(deg)
        dv = jnp.broadcast_to(dinv, (N, D))
        dv_ref[...] = dv
        g_ref[...] = dv * m_ref[...]
    return pl.pallas_call(
        body,
        out_shape=(jax.ShapeDtypeStruct((N, D), jnp.float32),
                   jax.ShapeDtypeStruct((N, D), jnp.float32)),
    )(degp, m1)


@jax.jit
def _tc_mid(acc1, m1, dv, b1_2, W2):
    def body(a_ref, m_ref, dv_ref, b_ref, w2_ref, g_ref, m2_ref):
        dv = dv_ref[...]
        conv = dv * (a_ref[0] + a_ref[1]) + dv * dv * m_ref[...] + b_ref[...]
        out1 = jnp.maximum(conv, 0.0)
        m2 = _mm_t(out1, w2_ref[...])
        m2_ref[...] = m2
        g_ref[...] = dv * m2
    return pl.pallas_call(
        body,
        out_shape=(jax.ShapeDtypeStruct((N, D), jnp.float32),
                   jax.ShapeDtypeStruct((N, D), jnp.float32)),
    )(acc1, m1, dv, b1_2, W2)


@jax.jit
def _tc_final(acc2, m2, dv, b2_2):
    def body(a_ref, m_ref, dv_ref, b_ref, o_ref):
        dv = dv_ref[...]
        h = dv * (a_ref[0] + a_ref[1]) + dv * dv * m_ref[...] + b_ref[...]
        nrm = jnp.sqrt(jnp.sum(h * h, axis=-1, keepdims=True))
        o_ref[...] = h / jnp.maximum(nrm, 1e-12)
    return pl.pallas_call(
        body, out_shape=jax.ShapeDtypeStruct((N, D), jnp.float32),
    )(acc2, m2, dv, b2_2)


def kernel(x, edge_index, W_pre, b_pre, W1, b1, W2, b2):
    src = edge_index[0].astype(jnp.int32)
    dst = edge_index[1].astype(jnp.int32)

    degp = _sc_degree(dst)
    m1 = _tc_pre(x, W_pre, b_pre.reshape(1, D), W1)
    g1, dv = _tc_norms(degp, m1)
    acc1 = _sc_conv(g1, src, dst)
    g2, m2 = _tc_mid(acc1, m1, dv, b1.reshape(1, D), W2)
    acc2 = _sc_conv(g2, src, dst)
    return _tc_final(acc2, m2, dv, b2.reshape(1, D))


# speedup vs baseline: 12.3630x; 12.3630x over previous
import functools

import jax
import jax.numpy as jnp
from jax import lax
from jax.experimental import pallas as pl
from jax.experimental.pallas import tpu as pltpu
from jax.experimental.pallas import tpu_sc as plsc

N = 10000
NP = 10240
E = 320000
D = 128

NC = 2
NS = 16
NW = NC * NS
C = 128
CHUNKS = E // C
TPW = -(-CHUNKS // NW)

_mesh = plsc.VectorSubcoreMesh(core_axis_name="c", subcore_axis_name="s")


@jax.jit
def _sc_conv(g, src, dst, zeros):
    @functools.partial(
        pl.kernel,
        out_type=jax.ShapeDtypeStruct((NC, NP, D), jnp.float32),
        mesh=_mesh,
        scratch_types=[
            pltpu.VMEM((C,), jnp.int32),
            pltpu.VMEM((C,), jnp.int32),
            pltpu.VMEM((C, D), jnp.float32),
            pltpu.VMEM_SHARED((NP, D), jnp.float32),
        ],
    )
    def k(g_hbm, src_hbm, dst_hbm, z_hbm, out_hbm, si_v, di_v, rows_v, acc_sh):
        c = lax.axis_index("c")
        s = lax.axis_index("s")
        w = s * NC + c

        @pl.when(s == 0)
        def _():
            pltpu.sync_copy(z_hbm, acc_sh)

        plsc.subcore_barrier()

        @pl.loop(0, TPW)
        def _(t):
            chunk = t * NW + w

            @pl.when(chunk < CHUNKS)
            def _():
                base = chunk * C
                pltpu.sync_copy(src_hbm.at[pl.ds(base, C)], si_v)
                pltpu.sync_copy(dst_hbm.at[pl.ds(base, C)], di_v)
                pltpu.sync_copy(g_hbm.at[si_v], rows_v)
                pltpu.sync_copy(rows_v, acc_sh.at[di_v], add=True)

        plsc.subcore_barrier()

        @pl.when(s == 0)
        def _():
            pltpu.sync_copy(acc_sh, out_hbm.at[c])

    return k(g, src, dst, zeros)


def _mm_t(a, w):
    return lax.dot_general(a, w, (((1,), (1,)), ((), ())),
                           preferred_element_type=jnp.float32)


@jax.jit
def _tc_pre(x, W_pre, b_pre2, W1):
    def body(x_ref, wp_ref, bp_ref, w1_ref, o_ref):
        h = _mm_t(x_ref[...], wp_ref[...]) + bp_ref[...]
        o_ref[...] = _mm_t(h, w1_ref[...])
    return pl.pallas_call(
        body, out_shape=jax.ShapeDtypeStruct((N, D), jnp.float32),
    )(x, W_pre, b_pre2, W1)


@jax.jit
def _tc_norms(degp, m1):
    def body(p_ref, m_ref, g_ref, dv_ref):
        deg = 1.0 + p_ref[0, :N, :1] + p_ref[1, :N, :1]
        dinv = lax.rsqrt(deg)
        dv = jnp.broadcast_to(dinv, (N, D))
        dv_ref[...] = dv
        g_ref[...] = dv * m_ref[...]
    return pl.pallas_call(
        body,
        out_shape=(jax.ShapeDtypeStruct((N, D), jnp.float32),
                   jax.ShapeDtypeStruct((N, D), jnp.float32)),
    )(degp, m1)


@jax.jit
def _tc_mid(acc1, m1, dv, b1_2, W2):
    def body(a_ref, m_ref, dv_ref, b_ref, w2_ref, g_ref, m2_ref):
        dv = dv_ref[...]
        conv = dv * (a_ref[0, :N] + a_ref[1, :N]) + dv * dv * m_ref[...] + b_ref[...]
        out1 = jnp.maximum(conv, 0.0)
        m2 = _mm_t(out1, w2_ref[...])
        m2_ref[...] = m2
        g_ref[...] = dv * m2
    return pl.pallas_call(
        body,
        out_shape=(jax.ShapeDtypeStruct((N, D), jnp.float32),
                   jax.ShapeDtypeStruct((N, D), jnp.float32)),
    )(acc1, m1, dv, b1_2, W2)


@jax.jit
def _tc_final(acc2, m2, dv, b2_2):
    def body(a_ref, m_ref, dv_ref, b_ref, o_ref):
        dv = dv_ref[...]
        h = dv * (a_ref[0, :N] + a_ref[1, :N]) + dv * dv * m_ref[...] + b_ref[...]
        nrm = jnp.sqrt(jnp.sum(h * h, axis=-1, keepdims=True))
        o_ref[...] = h / jnp.maximum(nrm, 1e-12)
    return pl.pallas_call(
        body, out_shape=jax.ShapeDtypeStruct((N, D), jnp.float32),
    )(acc2, m2, dv, b2_2)


def kernel(x, edge_index, W_pre, b_pre, W1, b1, W2, b2):
    src = edge_index[0].astype(jnp.int32)
    dst = edge_index[1].astype(jnp.int32)
    zeros = jnp.zeros((NP, D), jnp.float32)
    ones = jnp.ones((N, D), jnp.float32)

    degp = _sc_conv(ones, src, dst, zeros)
    m1 = _tc_pre(x, W_pre, b_pre.reshape(1, D), W1)
    g1, dv = _tc_norms(degp, m1)
    acc1 = _sc_conv(g1, src, dst, zeros)
    g2, m2 = _tc_mid(acc1, m1, dv, b1.reshape(1, D), W2)
    acc2 = _sc_conv(g2, src, dst, zeros)
    return _tc_final(acc2, m2, dv, b2.reshape(1, D))


# speedup vs baseline: 14.8720x; 1.2029x over previous
import functools

import jax
import jax.numpy as jnp
from jax import lax
from jax.experimental import pallas as pl
from jax.experimental.pallas import tpu as pltpu
from jax.experimental.pallas import tpu_sc as plsc

N = 10000
NP = 10240
E = 320000
D = 128

NC = 2
NS = 16
NW = NC * NS
C = 128
CHUNKS = E // C
TPW = -(-CHUNKS // NW)

_mesh = plsc.VectorSubcoreMesh(core_axis_name="c", subcore_axis_name="s")


@jax.jit
def _sc_conv(g, src, dst, zeros):
    @functools.partial(
        pl.kernel,
        out_type=jax.ShapeDtypeStruct((NC, NP, D), jnp.float32),
        mesh=_mesh,
        scratch_types=[
            pltpu.VMEM((C,), jnp.int32),
            pltpu.VMEM((C,), jnp.int32),
            pltpu.VMEM((C, D), jnp.float32),
            pltpu.VMEM_SHARED((NP, D), jnp.float32),
        ],
    )
    def k(g_hbm, src_hbm, dst_hbm, z_hbm, out_hbm, si_v, di_v, rows_v, acc_sh):
        c = lax.axis_index("c")
        s = lax.axis_index("s")
        w = s * NC + c

        @pl.when(s == 0)
        def _():
            pltpu.sync_copy(z_hbm, acc_sh)

        plsc.subcore_barrier()

        @pl.loop(0, TPW)
        def _(t):
            chunk = t * NW + w

            @pl.when(chunk < CHUNKS)
            def _():
                base = chunk * C
                pltpu.sync_copy(src_hbm.at[pl.ds(base, C)], si_v)
                pltpu.sync_copy(dst_hbm.at[pl.ds(base, C)], di_v)
                pltpu.sync_copy(g_hbm.at[si_v], rows_v)
                pltpu.sync_copy(rows_v, acc_sh.at[di_v], add=True)

        plsc.subcore_barrier()

        @pl.when(s == 0)
        def _():
            pltpu.sync_copy(acc_sh, out_hbm.at[c])

    return k(g, src, dst, zeros)


@jax.jit
def _sc_deg(ones_c, dst, zeros):
    @functools.partial(
        pl.kernel,
        out_type=jax.ShapeDtypeStruct((NC, NP, D), jnp.float32),
        mesh=_mesh,
        scratch_types=[
            pltpu.VMEM((C,), jnp.int32),
            pltpu.VMEM((C, D), jnp.float32),
            pltpu.VMEM_SHARED((NP, D), jnp.float32),
        ],
    )
    def k(ones_hbm, dst_hbm, z_hbm, out_hbm, di_v, rows_v, acc_sh):
        c = lax.axis_index("c")
        s = lax.axis_index("s")
        w = s * NC + c

        pltpu.sync_copy(ones_hbm, rows_v)

        @pl.when(s == 0)
        def _():
            pltpu.sync_copy(z_hbm, acc_sh)

        plsc.subcore_barrier()

        @pl.loop(0, TPW)
        def _(t):
            chunk = t * NW + w

            @pl.when(chunk < CHUNKS)
            def _():
                pltpu.sync_copy(dst_hbm.at[pl.ds(chunk * C, C)], di_v)
                pltpu.sync_copy(rows_v, acc_sh.at[di_v], add=True)

        plsc.subcore_barrier()

        @pl.when(s == 0)
        def _():
            pltpu.sync_copy(acc_sh, out_hbm.at[c])

    return k(ones_c, dst, zeros)


def _mm_t(a, w):
    return lax.dot_general(a, w, (((1,), (1,)), ((), ())),
                           preferred_element_type=jnp.float32)


@jax.jit
def _tc_pre(x, W_pre, b_pre2, W1):
    def body(x_ref, wp_ref, bp_ref, w1_ref, o_ref):
        h = _mm_t(x_ref[...], wp_ref[...]) + bp_ref[...]
        o_ref[...] = _mm_t(h, w1_ref[...])
    return pl.pallas_call(
        body, out_shape=jax.ShapeDtypeStruct((N, D), jnp.float32),
    )(x, W_pre, b_pre2, W1)


@jax.jit
def _tc_norms(degp, m1):
    def body(p_ref, m_ref, g_ref, dv_ref):
        deg = 1.0 + p_ref[0, :N, :1] + p_ref[1, :N, :1]
        dinv = lax.rsqrt(deg)
        dv = jnp.broadcast_to(dinv, (N, D))
        dv_ref[...] = dv
        g_ref[...] = dv * m_ref[...]
    return pl.pallas_call(
        body,
        out_shape=(jax.ShapeDtypeStruct((N, D), jnp.float32),
                   jax.ShapeDtypeStruct((N, D), jnp.float32)),
    )(degp, m1)


@jax.jit
def _tc_mid(acc1, m1, dv, b1_2, W2):
    def body(a_ref, m_ref, dv_ref, b_ref, w2_ref, g_ref, m2_ref):
        dv = dv_ref[...]
        conv = dv * (a_ref[0, :N] + a_ref[1, :N]) + dv * dv * m_ref[...] + b_ref[...]
        out1 = jnp.maximum(conv, 0.0)
        m2 = _mm_t(out1, w2_ref[...])
        m2_ref[...] = m2
        g_ref[...] = dv * m2
    return pl.pallas_call(
        body,
        out_shape=(jax.ShapeDtypeStruct((N, D), jnp.float32),
                   jax.ShapeDtypeStruct((N, D), jnp.float32)),
    )(acc1, m1, dv, b1_2, W2)


@jax.jit
def _tc_final(acc2, m2, dv, b2_2):
    def body(a_ref, m_ref, dv_ref, b_ref, o_ref):
        dv = dv_ref[...]
        h = dv * (a_ref[0, :N] + a_ref[1, :N]) + dv * dv * m_ref[...] + b_ref[...]
        nrm = jnp.sqrt(jnp.sum(h * h, axis=-1, keepdims=True))
        o_ref[...] = h / jnp.maximum(nrm, 1e-12)
    return pl.pallas_call(
        body, out_shape=jax.ShapeDtypeStruct((N, D), jnp.float32),
    )(acc2, m2, dv, b2_2)


def kernel(x, edge_index, W_pre, b_pre, W1, b1, W2, b2):
    src = edge_index[0].astype(jnp.int32)
    dst = edge_index[1].astype(jnp.int32)
    zeros = jnp.zeros((NP, D), jnp.float32)
    ones_c = jnp.ones((C, D), jnp.float32)

    degp = _sc_deg(ones_c, dst, zeros)
    m1 = _tc_pre(x, W_pre, b_pre.reshape(1, D), W1)
    g1, dv = _tc_norms(degp, m1)
    acc1 = _sc_conv(g1, src, dst, zeros)
    g2, m2 = _tc_mid(acc1, m1, dv, b1.reshape(1, D), W2)
    acc2 = _sc_conv(g2, src, dst, zeros)
    return _tc_final(acc2, m2, dv, b2.reshape(1, D))


# speedup vs baseline: 14.9332x; 1.0041x over previous
import functools

import jax
import jax.numpy as jnp
from jax import lax
from jax.experimental import pallas as pl
from jax.experimental.pallas import tpu as pltpu
from jax.experimental.pallas import tpu_sc as plsc

N = 10000
NP = 10240
E = 320000
D = 128

NC = 2
NS = 16
NW = NC * NS
C = 128
CHUNKS = E // C
TPW = -(-CHUNKS // NW)
RPS = NP // NS

_mesh = plsc.VectorSubcoreMesh(core_axis_name="c", subcore_axis_name="s")


@jax.jit
def _sc_conv(g, src, dst, zeros):
    @functools.partial(
        pl.kernel,
        out_type=jax.ShapeDtypeStruct((NC, NP, D), jnp.float32),
        mesh=_mesh,
        scratch_types=[
            pltpu.VMEM((C,), jnp.int32),
            pltpu.VMEM((C,), jnp.int32),
            pltpu.VMEM((C, D), jnp.float32),
            pltpu.VMEM_SHARED((NP, D), jnp.float32),
        ],
    )
    def k(g_hbm, src_hbm, dst_hbm, z_hbm, out_hbm, si_v, di_v, rows_v, acc_sh):
        c = lax.axis_index("c")
        s = lax.axis_index("s")
        w = s * NC + c

        pltpu.sync_copy(z_hbm.at[pl.ds(s * RPS, RPS)],
                        acc_sh.at[pl.ds(s * RPS, RPS)])
        plsc.subcore_barrier()

        @pl.loop(0, TPW)
        def _(t):
            chunk = t * NW + w

            @pl.when(chunk < CHUNKS)
            def _():
                base = chunk * C
                pltpu.sync_copy(src_hbm.at[pl.ds(base, C)], si_v)
                pltpu.sync_copy(dst_hbm.at[pl.ds(base, C)], di_v)
                pltpu.sync_copy(g_hbm.at[si_v], rows_v)
                pltpu.sync_copy(rows_v, acc_sh.at[di_v], add=True)

        plsc.subcore_barrier()
        pltpu.sync_copy(acc_sh.at[pl.ds(s * RPS, RPS)],
                        out_hbm.at[c, pl.ds(s * RPS, RPS)])

    return k(g, src, dst, zeros)


@jax.jit
def _sc_deg(ones_c, dst, zeros):
    @functools.partial(
        pl.kernel,
        out_type=jax.ShapeDtypeStruct((NC, NP, D), jnp.float32),
        mesh=_mesh,
        scratch_types=[
            pltpu.VMEM((C,), jnp.int32),
            pltpu.VMEM((C, D), jnp.float32),
            pltpu.VMEM_SHARED((NP, D), jnp.float32),
        ],
    )
    def k(ones_hbm, dst_hbm, z_hbm, out_hbm, di_v, rows_v, acc_sh):
        c = lax.axis_index("c")
        s = lax.axis_index("s")
        w = s * NC + c

        pltpu.sync_copy(ones_hbm, rows_v)

        pltpu.sync_copy(z_hbm.at[pl.ds(s * RPS, RPS)],
                        acc_sh.at[pl.ds(s * RPS, RPS)])
        plsc.subcore_barrier()

        @pl.loop(0, TPW)
        def _(t):
            chunk = t * NW + w

            @pl.when(chunk < CHUNKS)
            def _():
                pltpu.sync_copy(dst_hbm.at[pl.ds(chunk * C, C)], di_v)
                pltpu.sync_copy(rows_v, acc_sh.at[di_v], add=True)

        plsc.subcore_barrier()
        pltpu.sync_copy(acc_sh.at[pl.ds(s * RPS, RPS)],
                        out_hbm.at[c, pl.ds(s * RPS, RPS)])

    return k(ones_c, dst, zeros)


def _mm_t(a, w):
    return lax.dot_general(a, w, (((1,), (1,)), ((), ())),
                           preferred_element_type=jnp.float32)


@jax.jit
def _tc_pre(x, W_pre, b_pre2, W1):
    def body(x_ref, wp_ref, bp_ref, w1_ref, o_ref):
        h = _mm_t(x_ref[...], wp_ref[...]) + bp_ref[...]
        o_ref[...] = _mm_t(h, w1_ref[...])
    return pl.pallas_call(
        body, out_shape=jax.ShapeDtypeStruct((N, D), jnp.float32),
    )(x, W_pre, b_pre2, W1)


@jax.jit
def _tc_norms(degp, m1):
    def body(p_ref, m_ref, g_ref, dv_ref):
        deg = 1.0 + p_ref[0, :N, :1] + p_ref[1, :N, :1]
        dinv = lax.rsqrt(deg)
        dv = jnp.broadcast_to(dinv, (N, D))
        dv_ref[...] = dv
        g_ref[...] = dv * m_ref[...]
    return pl.pallas_call(
        body,
        out_shape=(jax.ShapeDtypeStruct((N, D), jnp.float32),
                   jax.ShapeDtypeStruct((N, D), jnp.float32)),
    )(degp, m1)


@jax.jit
def _tc_mid(acc1, m1, dv, b1_2, W2):
    def body(a_ref, m_ref, dv_ref, b_ref, w2_ref, g_ref, m2_ref):
        dv = dv_ref[...]
        conv = dv * (a_ref[0, :N] + a_ref[1, :N]) + dv * dv * m_ref[...] + b_ref[...]
        out1 = jnp.maximum(conv, 0.0)
        m2 = _mm_t(out1, w2_ref[...])
        m2_ref[...] = m2
        g_ref[...] = dv * m2
    return pl.pallas_call(
        body,
        out_shape=(jax.ShapeDtypeStruct((N, D), jnp.float32),
                   jax.ShapeDtypeStruct((N, D), jnp.float32)),
    )(acc1, m1, dv, b1_2, W2)


@jax.jit
def _tc_final(acc2, m2, dv, b2_2):
    def body(a_ref, m_ref, dv_ref, b_ref, o_ref):
        dv = dv_ref[...]
        h = dv * (a_ref[0, :N] + a_ref[1, :N]) + dv * dv * m_ref[...] + b_ref[...]
        nrm = jnp.sqrt(jnp.sum(h * h, axis=-1, keepdims=True))
        o_ref[...] = h / jnp.maximum(nrm, 1e-12)
    return pl.pallas_call(
        body, out_shape=jax.ShapeDtypeStruct((N, D), jnp.float32),
    )(acc2, m2, dv, b2_2)


def kernel(x, edge_index, W_pre, b_pre, W1, b1, W2, b2):
    src = edge_index[0].astype(jnp.int32)
    dst = edge_index[1].astype(jnp.int32)
    zeros = jnp.zeros((NP, D), jnp.float32)
    ones_c = jnp.ones((C, D), jnp.float32)

    degp = _sc_deg(ones_c, dst, zeros)
    m1 = _tc_pre(x, W_pre, b_pre.reshape(1, D), W1)
    g1, dv = _tc_norms(degp, m1)
    acc1 = _sc_conv(g1, src, dst, zeros)
    g2, m2 = _tc_mid(acc1, m1, dv, b1.reshape(1, D), W2)
    acc2 = _sc_conv(g2, src, dst, zeros)
    return _tc_final(acc2, m2, dv, b2.reshape(1, D))
